# baseline (speedup 1.0000x reference)
import jax
import jax.numpy as jnp
from jax import lax
from jax.experimental import pallas as pl
from jax.experimental.pallas import tpu as pltpu
from jax.experimental.pallas import tpu_sc as plsc

_NOBJ = 64
_V = 262144
_ROW = 1536
_ROWS_PER_OBJ = (_V * 3) // _ROW
_NC = 2
_NS = 16
_NW = _NC * _NS
_RPW = _ROWS_PER_OBJ // _NW


def _sc_body(obj_hbm, tbl_hbm, out_hbm, obj_v, idx_v, rows_v, sem):
    c = lax.axis_index("c")
    s = lax.axis_index("s")
    wid = s * _NC + c
    pltpu.sync_copy(obj_hbm, obj_v)
    obj = obj_v[...]
    idx = obj * _ROWS_PER_OBJ + wid * _RPW + lax.iota(jnp.int32, 16)
    idx_v[...] = idx
    pltpu.async_copy(tbl_hbm.at[idx_v], rows_v, sem).wait()
    pltpu.sync_copy(rows_v, out_hbm.at[pl.ds(wid * _RPW, _RPW)])


_gather = pl.kernel(
    _sc_body,
    out_type=jax.ShapeDtypeStruct((_ROWS_PER_OBJ, _ROW), jnp.float32),
    mesh=plsc.VectorSubcoreMesh(core_axis_name="c", subcore_axis_name="s"),
    scratch_types=[
        pltpu.VMEM((16,), jnp.int32),
        pltpu.VMEM((16,), jnp.int32),
        pltpu.VMEM((_RPW, _ROW), jnp.float32),
        pltpu.SemaphoreType.DMA,
    ],
)


def kernel(obj_id, weights):
    tbl = weights.reshape(_NOBJ * _ROWS_PER_OBJ, _ROW)
    obj16 = jnp.full((16,), obj_id, dtype=jnp.int32)
    out = _gather(obj16, tbl)
    return out.reshape(1, _V, 3)


# speedup vs baseline: 6.3393x; 6.3393x over previous
import jax
import jax.numpy as jnp
from jax import lax
from jax.experimental import pallas as pl
from jax.experimental.pallas import tpu as pltpu
from jax.experimental.pallas import tpu_sc as plsc

_NOBJ = 64
_V = 262144
_NC = 2
_NS = 16
_NW = _NC * _NS
_CH = _V // _NW


def _sc_body(obj_hbm, tbl_hbm, out_hbm, obj_v):
    c = lax.axis_index("c")
    s = lax.axis_index("s")
    wid = s * _NC + c
    pltpu.sync_copy(obj_hbm, obj_v)
    obj = obj_v[...][0]
    base = wid * _CH
    pltpu.sync_copy(tbl_hbm.at[obj, pl.ds(base, _CH)],
                    out_hbm.at[0, pl.ds(base, _CH)])


_gather = pl.kernel(
    _sc_body,
    out_type=jax.ShapeDtypeStruct((1, _V, 3), jnp.float32),
    mesh=plsc.VectorSubcoreMesh(core_axis_name="c", subcore_axis_name="s"),
    scratch_types=[
        pltpu.VMEM((16,), jnp.int32),
    ],
)


def kernel(obj_id, weights):
    obj16 = jnp.full((16,), obj_id, dtype=jnp.int32)
    return _gather(obj16, weights)


# speedup vs baseline: 6.4275x; 1.0139x over previous
import jax
import jax.numpy as jnp
from jax import lax
from jax.experimental import pallas as pl
from jax.experimental.pallas import tpu as pltpu
from jax.experimental.pallas import tpu_sc as plsc

_NOBJ = 64
_V = 262144
_NC = 2
_NS = 16
_NW = _NC * _NS
_CH = _V // _NW


def _sc_body(obj_hbm, tbl_hbm, out_hbm, obj_v):
    c = lax.axis_index("c")
    s = lax.axis_index("s")
    wid = s * _NC + c
    pltpu.sync_copy(obj_hbm, obj_v)
    obj = obj_v[...][0]

    @pl.when(wid == 0)
    def _():
        pltpu.sync_copy(tbl_hbm.at[obj], out_hbm.at[0])


_gather = pl.kernel(
    _sc_body,
    out_type=jax.ShapeDtypeStruct((1, _V, 3), jnp.float32),
    mesh=plsc.VectorSubcoreMesh(core_axis_name="c", subcore_axis_name="s"),
    scratch_types=[
        pltpu.VMEM((16,), jnp.int32),
    ],
)


def kernel(obj_id, weights):
    obj16 = jnp.full((16,), obj_id, dtype=jnp.int32)
    return _gather(obj16, weights)
